# baseline (speedup 1.0000x reference)
import functools

import jax
import jax.numpy as jnp
from jax import lax
from jax.experimental import pallas as pl
from jax.experimental.pallas import tpu as pltpu
from jax.experimental.pallas import tpu_sc as plsc

_NC = 2
_NS = 16
_NW = _NC * _NS
_C = 128


def _mesh():
    return plsc.VectorSubcoreMesh(core_axis_name="c", subcore_axis_name="s")


def _qrsqrt(x):
    i = lax.bitcast_convert_type(x, jnp.int32)
    i = 0x5F3759DF - lax.shift_right_arithmetic(i, 1)
    y = lax.bitcast_convert_type(i, jnp.float32)
    for _ in range(3):
        y = y * (1.5 - 0.5 * x * y * y)
    return y


def _make_deg_fn(n_acc, k):
    stripe = n_acc // _NS

    @functools.partial(
        pl.kernel,
        out_type=jax.ShapeDtypeStruct((n_acc,), jnp.float32),
        mesh=_mesh(),
        scratch_types=[
            pltpu.VMEM((k, _C), jnp.int32),
            pltpu.VMEM((_C,), jnp.float32),
            pltpu.VMEM((stripe,), jnp.float32),
            pltpu.VMEM_SHARED((n_acc,), jnp.float32),
        ],
    )
    def deg_fn(dst_hbm, zeros_hbm, out_hbm, didx, ones_v, dbuf, shared):
        cid = lax.axis_index("c")
        sid = lax.axis_index("s")
        r0 = pl.multiple_of(sid * stripe, stripe)
        for i in range(_C // 16):
            ones_v[pl.ds(i * 16, 16)] = jnp.ones((16,), jnp.float32)
        pltpu.sync_copy(zeros_hbm.at[pl.ds(r0, stripe)],
                        shared.at[pl.ds(r0, stripe)])
        plsc.subcore_barrier()
        for half in range(2):
            pltpu.sync_copy(dst_hbm.at[sid + half * _NS], didx)

            def body(j, carry):
                pltpu.sync_copy(ones_v, shared.at[didx.at[j]], add=True)
                return carry

            lax.fori_loop(0, k, body, 0)
        plsc.subcore_barrier()

        @pl.when(cid == 0)
        def _():
            pltpu.sync_copy(shared.at[pl.ds(r0, stripe)], dbuf)
            def rs(i, carry):
                c = dbuf[pl.ds(i * 16, 16)]
                dbuf[pl.ds(i * 16, 16)] = _qrsqrt(c + 1.0)
                return carry
            lax.fori_loop(0, stripe // 16, rs, 0)
            pltpu.sync_copy(dbuf, out_hbm.at[pl.ds(r0, stripe)])

    return deg_fn


def _make_mega_fn(n_acc, k, half_w):
    stripe = n_acc // _NS
    f32 = jnp.float32

    @functools.partial(
        pl.kernel,
        out_type=[
            jax.ShapeDtypeStruct((_NC * n_acc, half_w), f32),
            jax.ShapeDtypeStruct((_NC * n_acc, half_w), f32),
        ],
        mesh=_mesh(),
        scratch_types=[
            pltpu.VMEM((k, _C), jnp.int32),
            pltpu.VMEM((k, _C), jnp.int32),
            pltpu.VMEM((_C, half_w), f32),
            pltpu.VMEM((stripe, half_w), f32),
            pltpu.VMEM((stripe, half_w), f32),
            pltpu.VMEM((32,), f32),
            pltpu.VMEM((n_acc,), f32),
            pltpu.VMEM_SHARED((n_acc, half_w), f32),
            pltpu.SemaphoreType.DMA,
        ],
        compiler_params=pltpu.CompilerParams(use_tc_tiling_on_sc=False,
                                             needs_layout_passes=False),
    )
    def mega_fn(srcadj_hbm, dst_hbm, g1_hbm, dinv_hbm, zeros_hbm, b1p_hbm,
                agg2_hbm, g2_hbm,
                sidx, didx, hbuf, abuf, gbuf, bbuf, dinv_t, acc, sem):
        cid = lax.axis_index("c")
        sid = lax.axis_index("s")
        r0 = pl.multiple_of(sid * stripe, stripe)
        gofs = pl.multiple_of(cid * n_acc + r0, stripe)

        pltpu.sync_copy(dinv_hbm, dinv_t)
        pltpu.sync_copy(zeros_hbm.at[pl.ds(r0, stripe)],
                        acc.at[pl.ds(r0, stripe)])
        pltpu.sync_copy(b1p_hbm, bbuf)
        plsc.subcore_barrier()

        def edge_pass(table_hbm):
            for half in range(2):
                w = sid + half * _NS
                pltpu.sync_copy(srcadj_hbm.at[cid * _NW + w], sidx)
                pltpu.sync_copy(dst_hbm.at[w], didx)

                def body(j, carry):
                    pltpu.async_copy(table_hbm.at[sidx.at[j]], hbuf,
                                     sem).wait()
                    pltpu.sync_copy(hbuf, acc.at[didx.at[j]], add=True)
                    return carry

                lax.fori_loop(0, k, body, 0)

        edge_pass(g1_hbm)
        plsc.subcore_barrier()

        iot = lax.iota(jnp.int32, 16)
        base = lax.shift_right_logical(iot, 3)
        colp = lax.bitwise_and(iot, 7)
        bvec = bbuf[pl.ds(cid * 16, 16)]
        pltpu.sync_copy(acc.at[pl.ds(r0, stripe)], abuf)
        pltpu.sync_copy(g1_hbm.at[pl.ds(gofs, stripe)], gbuf)

        def mid(p, carry):
            rl = base + 2 * p
            a = plsc.load_gather(abuf, [rl, colp])
            g = plsc.load_gather(gbuf, [rl, colp])
            d = plsc.load_gather(dinv_t, [base + (r0 + 2 * p)])
            h2 = jnp.maximum((a + g) * d + bvec, 0.0)
            plsc.store_scatter(gbuf, [rl, colp], h2 * d)
            return carry

        lax.fori_loop(0, stripe // 2, mid, 0)
        pltpu.sync_copy(gbuf, g2_hbm.at[pl.ds(gofs, stripe)])
        pltpu.sync_copy(zeros_hbm.at[pl.ds(r0, stripe)],
                        acc.at[pl.ds(r0, stripe)])
        plsc.subcore_barrier()

        edge_pass(g2_hbm)
        plsc.subcore_barrier()

        pltpu.sync_copy(acc.at[pl.ds(r0, stripe)], abuf)

        def fin(p, carry):
            rl = base + 2 * p
            a = plsc.load_gather(abuf, [rl, colp])
            g = plsc.load_gather(gbuf, [rl, colp])
            d = plsc.load_gather(dinv_t, [base + (r0 + 2 * p)])
            plsc.store_scatter(abuf, [rl, colp], (a + g) * d)
            return carry

        lax.fori_loop(0, stripe // 2, fin, 0)
        pltpu.sync_copy(abuf, agg2_hbm.at[pl.ds(gofs, stripe)])

    return mega_fn


def _layer1_body(x_ref, w1_ref, dinv_ref, g1_ref):
    h = jnp.dot(x_ref[...], w1_ref[...], preferred_element_type=jnp.float32)
    g = h * dinv_ref[...]
    hw = g.shape[1] // 2
    g1_ref[...] = jnp.concatenate([g[:, :hw], g[:, hw:]], axis=0)


def _out_body(a0_ref, a1_ref, w2_ref, b2_ref, out_ref):
    agg = jnp.concatenate([a0_ref[...], a1_ref[...]], axis=1)
    out_ref[...] = (
        jnp.dot(agg, w2_ref[...], preferred_element_type=jnp.float32)
        + b2_ref[...]
    )


def _tc(body, out_shape, *args):
    return pl.pallas_call(body, out_shape=out_shape)(*args)


def kernel(x, edge_index, W1, b1, W2, b2):
    f32 = jnp.float32
    n, _ = x.shape
    hid = W1.shape[1]
    d_out = W2.shape[1]
    e = edge_index.shape[1]

    n_acc = (n // (2 * _C) + 1) * 2 * _C
    k = (e + _NW * _C - 1) // (_NW * _C)
    e_pad = _NW * _C * k
    pad_n = e_pad - e

    src = edge_index[0]
    dst = edge_index[1]
    if pad_n:
        ar = jnp.arange(pad_n, dtype=edge_index.dtype)
        src = jnp.concatenate([src, ar % n])
        dst = jnp.concatenate([dst, n + ar % (n_acc - n)])
    src_g = src.reshape(_NW, k, _C)
    dst_g = dst.reshape(_NW, k, _C)
    src_adj = jnp.stack([src_g, src_g + n_acc]).reshape(_NC * _NW, k, _C)

    half_w = hid // 2
    zeros1 = jnp.zeros((n_acc,), f32)
    zeros8 = jnp.zeros((n_acc, half_w), f32)
    b1_pat = jnp.concatenate(
        [jnp.tile(b1[:half_w], 2), jnp.tile(b1[half_w:], 2)])
    x_pad = jnp.pad(x, ((0, n_acc - n), (0, 0)))

    deg_fn = _make_deg_fn(n_acc, k)
    mega_fn = _make_mega_fn(n_acc, k, half_w)

    dinv = deg_fn(dst_g, zeros1)
    g1 = _tc(_layer1_body, jax.ShapeDtypeStruct((_NC * n_acc, half_w), f32),
             x_pad, W1, dinv.reshape(n_acc, 1))
    agg2, _ = mega_fn(src_adj, dst_g, g1, dinv, zeros8, b1_pat)
    out = _tc(_out_body, jax.ShapeDtypeStruct((n_acc, d_out), f32),
              agg2[:n_acc], agg2[n_acc:], W2, b2.reshape(1, d_out))
    return out[:n]


# speedup vs baseline: 1.3483x; 1.3483x over previous
import functools

import jax
import jax.numpy as jnp
from jax import lax
from jax.experimental import pallas as pl
from jax.experimental.pallas import tpu as pltpu
from jax.experimental.pallas import tpu_sc as plsc

_NC = 2
_NS = 16
_NW = _NC * _NS
_C = 128


def _mesh():
    return plsc.VectorSubcoreMesh(core_axis_name="c", subcore_axis_name="s")


def _qrsqrt(x):
    i = lax.bitcast_convert_type(x, jnp.int32)
    i = 0x5F3759DF - lax.shift_right_arithmetic(i, 1)
    y = lax.bitcast_convert_type(i, jnp.float32)
    for _ in range(3):
        y = y * (1.5 - 0.5 * x * y * y)
    return y


def _make_pass1_fn(n_acc, k, hid):
    stripe = n_acc // _NS

    @functools.partial(
        pl.kernel,
        out_type=[
            jax.ShapeDtypeStruct((_NC, n_acc, hid), jnp.float32),
            jax.ShapeDtypeStruct((n_acc,), jnp.float32),
        ],
        mesh=_mesh(),
        scratch_types=[
            pltpu.VMEM((k, _C), jnp.int32),
            pltpu.VMEM((k, _C), jnp.int32),
            pltpu.VMEM((_C,), jnp.float32),
            pltpu.VMEM((stripe,), jnp.float32),
            pltpu.VMEM((n_acc,), jnp.float32),
            pltpu.VMEM((16,), jnp.float32),
            pltpu.VMEM((_C, hid), jnp.float32),
            pltpu.VMEM_SHARED((n_acc,), jnp.float32),
            pltpu.VMEM_SHARED((n_acc,), jnp.float32),
            pltpu.VMEM_SHARED((n_acc, hid), jnp.float32),
            pltpu.SemaphoreType.DMA,
        ],
        compiler_params=pltpu.CompilerParams(use_tc_tiling_on_sc=False,
                                             needs_layout_passes=False),
    )
    def pass1_fn(src_hbm, dst_hbm, h1_hbm, zeros1_hbm, zeros2_hbm,
                 p_hbm, dinv_hbm,
                 sidx, didx, ones_v, dbuf, dinv_t, dstage, hbuf,
                 cnt_sh, dinv_sh, acc, sem):
        cid = lax.axis_index("c")
        sid = lax.axis_index("s")
        w = cid * _NS + sid
        r0 = pl.multiple_of(sid * stripe, stripe)

        for i in range(_C // 16):
            ones_v[pl.ds(i * 16, 16)] = jnp.ones((16,), jnp.float32)
        pltpu.sync_copy(zeros1_hbm.at[pl.ds(r0, stripe)],
                        cnt_sh.at[pl.ds(r0, stripe)])
        pltpu.sync_copy(zeros2_hbm.at[pl.ds(r0, stripe)],
                        acc.at[pl.ds(r0, stripe)])
        plsc.subcore_barrier()
        for half in range(2):
            pltpu.sync_copy(dst_hbm.at[sid + half * _NS], didx)

            def cbody(j, carry):
                pltpu.sync_copy(ones_v, cnt_sh.at[didx.at[j]], add=True)
                return carry

            lax.fori_loop(0, k, cbody, 0)
        plsc.subcore_barrier()

        pltpu.sync_copy(cnt_sh.at[pl.ds(r0, stripe)], dbuf)

        def rbody(i, carry):
            c = dbuf[pl.ds(i * 16, 16)]
            dbuf[pl.ds(i * 16, 16)] = _qrsqrt(c + 1.0)
            return carry

        lax.fori_loop(0, stripe // 16, rbody, 0)
        pltpu.sync_copy(dbuf, dinv_sh.at[pl.ds(r0, stripe)])

        @pl.when(cid == 0)
        def _():
            pltpu.sync_copy(dbuf, dinv_hbm.at[pl.ds(r0, stripe)])

        plsc.subcore_barrier()
        pltpu.sync_copy(dinv_sh, dinv_t)

        pltpu.sync_copy(src_hbm.at[w], sidx)
        pltpu.sync_copy(dst_hbm.at[w], didx)
        iot = lax.iota(jnp.int32, 16)
        zero16 = lax.bitwise_and(iot, 0)

        def ebody(j, carry):
            pltpu.async_copy(h1_hbm.at[sidx.at[j]], hbuf, sem).wait()
            for q in range(_C // 16):
                sv = sidx[j, pl.ds(q * 16, 16)]
                d16 = plsc.load_gather(dinv_t, [sv])
                for m in range(16):
                    dm = lax.gather(
                        d16, (zero16 + m)[:, None],
                        lax.GatherDimensionNumbers(
                            offset_dims=(), collapsed_slice_dims=(0,),
                            start_index_map=(0,)),
                        slice_sizes=(1,),
                        mode=lax.GatherScatterMode.PROMISE_IN_BOUNDS)
                    r = q * 16 + m
                    hbuf[r, :] = hbuf[r, :] * dm
            pltpu.sync_copy(hbuf, acc.at[didx.at[j]], add=True)
            return carry

        lax.fori_loop(0, k, ebody, 0)
        plsc.subcore_barrier()
        pltpu.sync_copy(acc.at[pl.ds(r0, stripe)],
                        p_hbm.at[cid, pl.ds(r0, stripe)])

    return pass1_fn


def _make_scatter_fn(n_acc, k, hid):
    stripe = n_acc // _NS

    @functools.partial(
        pl.kernel,
        out_type=jax.ShapeDtypeStruct((_NC, n_acc, hid), jnp.float32),
        mesh=_mesh(),
        scratch_types=[
            pltpu.VMEM((k, _C), jnp.int32),
            pltpu.VMEM((k, _C), jnp.int32),
            pltpu.VMEM((_C, hid), jnp.float32),
            pltpu.VMEM_SHARED((n_acc, hid), jnp.float32),
            pltpu.SemaphoreType.DMA,
        ],
        compiler_params=pltpu.CompilerParams(use_tc_tiling_on_sc=False),
    )
    def scatter_fn(src_hbm, dst_hbm, table_hbm, zeros_hbm, out_hbm,
                   sidx, didx, rows, shared, sem):
        cid = lax.axis_index("c")
        sid = lax.axis_index("s")
        w = cid * _NS + sid
        r0 = pl.multiple_of(sid * stripe, stripe)
        pltpu.sync_copy(zeros_hbm.at[pl.ds(r0, stripe)],
                        shared.at[pl.ds(r0, stripe)])
        pltpu.sync_copy(src_hbm.at[w], sidx)
        pltpu.sync_copy(dst_hbm.at[w], didx)
        plsc.subcore_barrier()

        def body(j, carry):
            pltpu.async_copy(table_hbm.at[sidx.at[j]], rows, sem).wait()
            pltpu.sync_copy(rows, shared.at[didx.at[j]], add=True)
            return carry

        lax.fori_loop(0, k, body, 0)
        plsc.subcore_barrier()
        pltpu.sync_copy(shared.at[pl.ds(r0, stripe)],
                        out_hbm.at[cid, pl.ds(r0, stripe)])

    return scatter_fn


def _layer1_body(x_ref, w1_ref, h1_ref):
    h1_ref[...] = jnp.dot(x_ref[...], w1_ref[...],
                          preferred_element_type=jnp.float32)


def _mid_body(p0_ref, p1_ref, h1_ref, dinv_ref, b1_ref, g2_ref):
    d = dinv_ref[...]
    g1 = h1_ref[...] * d
    agg = (p0_ref[...] + p1_ref[...] + g1) * d
    h2 = jnp.maximum(agg + b1_ref[...], 0.0)
    g2_ref[...] = h2 * d


def _out_body(p0_ref, p1_ref, g2_ref, dinv_ref, w2_ref, b2_ref, out_ref):
    agg = (p0_ref[...] + p1_ref[...] + g2_ref[...]) * dinv_ref[...]
    out_ref[...] = (
        jnp.dot(agg, w2_ref[...], preferred_element_type=jnp.float32)
        + b2_ref[...]
    )


def _tc(body, out_shape, *args):
    return pl.pallas_call(body, out_shape=out_shape)(*args)


def kernel(x, edge_index, W1, b1, W2, b2):
    f32 = jnp.float32
    n, _ = x.shape
    hid = W1.shape[1]
    d_out = W2.shape[1]
    e = edge_index.shape[1]

    n_acc = (n // (2 * _C) + 1) * 2 * _C
    k = (e + _NW * _C - 1) // (_NW * _C)
    e_pad = _NW * _C * k
    pad_n = e_pad - e

    src = edge_index[0]
    dst = edge_index[1]
    if pad_n:
        ar = jnp.arange(pad_n, dtype=edge_index.dtype)
        src = jnp.concatenate([src, ar % n])
        dst = jnp.concatenate([dst, n + ar % (n_acc - n)])
    src_g = src.reshape(_NW, k, _C)
    dst_g = dst.reshape(_NW, k, _C)

    zeros1 = jnp.zeros((n_acc,), f32)
    zeros2 = jnp.zeros((n_acc, hid), f32)

    pass1_fn = _make_pass1_fn(n_acc, k, hid)
    scat_fn = _make_scatter_fn(n_acc, k, hid)

    h1 = _tc(_layer1_body, jax.ShapeDtypeStruct((n, hid), f32), x, W1)
    p1, dinv = pass1_fn(src_g, dst_g, h1, zeros1, zeros2)
    dinv_c = dinv[:n].reshape(n, 1)
    g2 = _tc(_mid_body, jax.ShapeDtypeStruct((n, hid), f32),
             p1[0, :n], p1[1, :n], h1, dinv_c, b1.reshape(1, hid))
    p2 = scat_fn(src_g, dst_g, g2, zeros2)
    out = _tc(_out_body, jax.ShapeDtypeStruct((n, d_out), f32),
              p2[0, :n], p2[1, :n], g2, dinv_c, W2, b2.reshape(1, d_out))
    return out


# speedup vs baseline: 1.4405x; 1.0683x over previous
import functools

import jax
import jax.numpy as jnp
from jax import lax
from jax.experimental import pallas as pl
from jax.experimental.pallas import tpu as pltpu
from jax.experimental.pallas import tpu_sc as plsc

_NC = 2
_NS = 16
_NW = _NC * _NS
_C = 128


def _mesh():
    return plsc.VectorSubcoreMesh(core_axis_name="c", subcore_axis_name="s")


def _make_deg_fn(n_acc, k):
    stripe = n_acc // _NS

    @functools.partial(
        pl.kernel,
        out_type=jax.ShapeDtypeStruct((_NC * n_acc,), jnp.float32),
        mesh=_mesh(),
        scratch_types=[
            pltpu.VMEM((k, _C), jnp.int32),
            pltpu.VMEM((_C,), jnp.float32),
            pltpu.VMEM_SHARED((n_acc,), jnp.float32),
        ],
    )
    def deg_fn(dst_hbm, zeros_hbm, out_hbm, didx, ones_v, shared):
        cid = lax.axis_index("c")
        sid = lax.axis_index("s")
        w = cid * _NS + sid
        r0 = pl.multiple_of(sid * stripe, stripe)
        pltpu.sync_copy(dst_hbm.at[w], didx)
        for i in range(_C // 16):
            ones_v[pl.ds(i * 16, 16)] = jnp.ones((16,), jnp.float32)
        pltpu.sync_copy(zeros_hbm.at[pl.ds(r0, stripe)],
                        shared.at[pl.ds(r0, stripe)])
        plsc.subcore_barrier()

        def body(j, carry):
            pltpu.sync_copy(ones_v, shared.at[didx.at[j]], add=True)
            return carry

        lax.fori_loop(0, k, body, 0)
        plsc.subcore_barrier()
        ofs = pl.multiple_of(cid * n_acc + r0, stripe)
        pltpu.sync_copy(shared.at[pl.ds(r0, stripe)],
                        out_hbm.at[pl.ds(ofs, stripe)])

    return deg_fn


def _make_scatter_fn(n_acc, k, hid):
    stripe = n_acc // _NS

    @functools.partial(
        pl.kernel,
        out_type=jax.ShapeDtypeStruct((_NC, n_acc, hid), jnp.float32),
        mesh=_mesh(),
        scratch_types=[
            pltpu.VMEM((k, _C), jnp.int32),
            pltpu.VMEM((k, _C), jnp.int32),
            pltpu.VMEM((_C, hid), jnp.float32),
            pltpu.VMEM_SHARED((n_acc, hid), jnp.float32),
            pltpu.SemaphoreType.DMA,
        ],
        compiler_params=pltpu.CompilerParams(use_tc_tiling_on_sc=False),
    )
    def scatter_fn(src_hbm, dst_hbm, table_hbm, zeros_hbm, out_hbm,
                   sidx, didx, rows, shared, sem):
        cid = lax.axis_index("c")
        sid = lax.axis_index("s")
        w = cid * _NS + sid
        r0 = pl.multiple_of(sid * stripe, stripe)
        pltpu.sync_copy(zeros_hbm.at[pl.ds(r0, stripe)],
                        shared.at[pl.ds(r0, stripe)])
        pltpu.sync_copy(src_hbm.at[w], sidx)
        pltpu.sync_copy(dst_hbm.at[w], didx)
        plsc.subcore_barrier()

        def body(j, carry):
            pltpu.async_copy(table_hbm.at[sidx.at[j]], rows, sem).wait()
            pltpu.sync_copy(rows, shared.at[didx.at[j]], add=True)
            return carry

        lax.fori_loop(0, k, body, 0)
        plsc.subcore_barrier()
        pltpu.sync_copy(shared.at[pl.ds(r0, stripe)],
                        out_hbm.at[cid, pl.ds(r0, stripe)])

    return scatter_fn


def _dinv_body(c0_ref, c1_ref, dinv_ref):
    cnt = c0_ref[...] + c1_ref[...]
    dinv_ref[...] = lax.rsqrt(cnt + 1.0)


def _layer1_body(x_ref, w1_ref, dinv_ref, g1_ref):
    h = jnp.dot(x_ref[...], w1_ref[...], preferred_element_type=jnp.float32)
    g1_ref[...] = h * dinv_ref[...]


def _mid_body(p0_ref, p1_ref, g1_ref, dinv_ref, b1_ref, g2_ref):
    agg = (p0_ref[...] + p1_ref[...] + g1_ref[...]) * dinv_ref[...]
    h2 = jnp.maximum(agg + b1_ref[...], 0.0)
    g2_ref[...] = h2 * dinv_ref[...]


def _out_body(p0_ref, p1_ref, g2_ref, dinv_ref, w2_ref, b2_ref, out_ref):
    agg = (p0_ref[...] + p1_ref[...] + g2_ref[...]) * dinv_ref[...]
    out_ref[...] = (
        jnp.dot(agg, w2_ref[...], preferred_element_type=jnp.float32)
        + b2_ref[...]
    )


def _tc(body, out_shape, *args):
    return pl.pallas_call(body, out_shape=out_shape)(*args)


def kernel(x, edge_index, W1, b1, W2, b2):
    f32 = jnp.float32
    n, _ = x.shape
    hid = W1.shape[1]
    d_out = W2.shape[1]
    e = edge_index.shape[1]

    n_acc = (n // (2 * _C) + 1) * 2 * _C
    k = (e + _NW * _C - 1) // (_NW * _C)
    e_pad = _NW * _C * k
    pad_n = e_pad - e

    src = edge_index[0]
    dst = edge_index[1]
    if pad_n:
        ar = jnp.arange(pad_n, dtype=edge_index.dtype)
        src = jnp.concatenate([src, ar % n])
        dst = jnp.concatenate([dst, n + ar % (n_acc - n)])
    src_g = src.reshape(_NW, k, _C)
    dst_g = dst.reshape(_NW, k, _C)

    zeros1 = jnp.zeros((n_acc,), f32)
    zeros2 = jnp.zeros((n_acc, hid), f32)

    deg_fn = _make_deg_fn(n_acc, k)
    scat_fn = _make_scatter_fn(n_acc, k, hid)

    cnt = deg_fn(dst_g, zeros1).reshape(_NC, n_acc)
    dinv = _tc(_dinv_body, jax.ShapeDtypeStruct((1, n), f32),
               cnt[0:1, :n], cnt[1:2, :n])
    dinv_c = dinv.reshape(n, 1)

    g1 = _tc(_layer1_body, jax.ShapeDtypeStruct((n, hid), f32),
             x, W1, dinv_c)
    p1 = scat_fn(src_g, dst_g, g1, zeros2)
    g2 = _tc(_mid_body, jax.ShapeDtypeStruct((n, hid), f32),
             p1[0, :n], p1[1, :n], g1, dinv_c, b1.reshape(1, hid))
    p2 = scat_fn(src_g, dst_g, g2, zeros2)
    out = _tc(_out_body, jax.ShapeDtypeStruct((n, d_out), f32),
              p2[0, :n], p2[1, :n], g2, dinv_c, W2, b2.reshape(1, d_out))
    return out
